# baseline (speedup 1.0000x reference)
import functools

import jax
import jax.numpy as jnp
from jax import lax
from jax.experimental import pallas as pl
from jax.experimental.pallas import tpu as pltpu
from jax.experimental.pallas import tpu_sc as plsc

N = 1000000
L = 16
NW = 32
EW0 = 31248
EW31 = 31312
CHUNK = 1024
KCH = 31

_DNUMS = lax.GatherDimensionNumbers(
    offset_dims=(), collapsed_slice_dims=(0,), start_index_map=(0,))


def _ple_kernel(x_hbm, coef_hbm, out_hbm, coef_v, x_v, out_v):
    wid = lax.axis_index("s") * 2 + lax.axis_index("c")
    base = wid * EW0
    ew = jnp.where(wid == NW - 1, EW31, EW0)

    pltpu.sync_copy(coef_hbm, coef_v)
    a = coef_v[pl.ds(0, L)]
    b = coef_v[pl.ds(L, L)]
    lo = coef_v[pl.ds(2 * L, L)]
    hi = coef_v[pl.ds(3 * L, L)]

    def chunk_body(k, _):
        off = jnp.minimum(k * CHUNK, ew - CHUNK)
        pltpu.sync_copy(x_hbm.at[pl.ds(base + off, CHUNK)], x_v)

        def group_body(g, _):
            gbase = g * L
            xv = x_v[pl.ds(gbase, L)]
            for e in range(L):
                idx = jnp.full((L, 1), e, dtype=jnp.int32)
                xe = lax.gather(
                    xv, idx, _DNUMS, slice_sizes=(1,),
                    mode=lax.GatherScatterMode.PROMISE_IN_BOUNDS)
                enc = jnp.minimum(jnp.maximum(xe * a + b, lo), hi)
                out_v[pl.ds((gbase + e) * L, L)] = enc
            return 0

        lax.fori_loop(0, CHUNK // L, group_body, 0)
        pltpu.sync_copy(out_v, out_hbm.at[pl.ds((base + off) * L, CHUNK * L)])
        return 0

    lax.fori_loop(0, KCH, chunk_body, 0)


@jax.jit
def _ple(xf, coef):
    mesh = plsc.VectorSubcoreMesh(core_axis_name="c", subcore_axis_name="s")
    f = functools.partial(
        pl.kernel,
        mesh=mesh,
        out_type=jax.ShapeDtypeStruct((N * L,), jnp.float32),
        scratch_types=[
            pltpu.VMEM((4 * L,), jnp.float32),
            pltpu.VMEM((CHUNK,), jnp.float32),
            pltpu.VMEM((CHUNK * L,), jnp.float32),
        ],
    )(_ple_kernel)
    return f(xf, coef)


def kernel(x, bins):
    n_bins = bins.shape[0]
    lk = jnp.concatenate([bins, jnp.array([-1.0], dtype=bins.dtype)])
    prev = lk[:n_bins]
    nxt = lk[1 : n_bins + 1]
    a = 1.0 / (nxt - prev)
    b = -prev * a
    j = jnp.arange(n_bins)
    neg_inf = jnp.float32(-jnp.inf)
    pos_inf = jnp.float32(jnp.inf)
    lo = jnp.where(j == 0, neg_inf, jnp.zeros((n_bins,), jnp.float32))
    hi = jnp.ones((n_bins,), jnp.float32)
    last_pos = a[n_bins - 1] > 0
    lo = lo.at[n_bins - 1].set(jnp.where(last_pos, 0.0, neg_inf))
    hi = hi.at[n_bins - 1].set(jnp.where(last_pos, pos_inf, 0.0))
    coef = jnp.concatenate([a, b, lo, hi]).astype(jnp.float32)
    out = _ple(x.reshape(-1), coef)
    return out.reshape(-1, 1, n_bins)
